# baseline (speedup 1.0000x reference)
import jax
import jax.numpy as jnp
from jax import lax
from jax.experimental import pallas as pl
from jax.experimental.pallas import tpu as pltpu
from jax.experimental.pallas import tpu_sc as plsc

N_NODES = 10000
N_PAD = 10240
N_EDGES = 320000
D = 128
H = 64

ROWS = 1000
NS = 16
CHUNK = 40
EDGES_PER_SUB = N_EDGES // NS
ROWS_PER_SUB = N_PAD // NS

_HI = jax.lax.Precision.HIGHEST


def _node_body(x_ref, pW1_ref, pb1_ref, pW2_ref, pb2_ref,
               fW1_ref, fb1_ref, fW2_ref, fb2_ref,
               cW1ab_ref, cb1ab_ref,
               S_ref, C_ref, F_ref, AB_ref):
    xb = x_ref[...]
    hp = jnp.tanh(jnp.dot(xb, pW1_ref[...], precision=_HI,
                          preferred_element_type=jnp.float32) + pb1_ref[...])
    phases = jnp.dot(hp, pW2_ref[...], precision=_HI,
                     preferred_element_type=jnp.float32) + pb2_ref[...]
    S_ref[...] = jnp.sin(phases)
    C_ref[...] = jnp.cos(phases)
    hf = jnp.maximum(jnp.dot(xb, fW1_ref[...], precision=_HI,
                             preferred_element_type=jnp.float32) + fb1_ref[...], 0.0)
    F_ref[...] = jnp.dot(hf, fW2_ref[...], precision=_HI,
                         preferred_element_type=jnp.float32) + fb2_ref[...]
    AB_ref[...] = jnp.dot(xb, cW1ab_ref[...], precision=_HI,
                          preferred_element_type=jnp.float32) + cb1ab_ref[...]


def _node_precompute(x, pW1, pb1, pW2, pb2, fW1, fb1, fW2, fb2, cW1ab, cb1ab):
    n_blocks = N_NODES // ROWS
    full = lambda s: pl.BlockSpec(s, lambda i: (0, 0))
    row = lambda w: pl.BlockSpec((ROWS, w), lambda i: (i, 0))
    return pl.pallas_call(
        _node_body,
        grid=(n_blocks,),
        in_specs=[row(D), full((D, H)), full((1, H)), full((H, D)), full((1, D)),
                  full((D, H)), full((1, H)), full((H, D)), full((1, D)),
                  full((D, D)), full((1, D))],
        out_specs=[row(D), row(D), row(D), row(D)],
        out_shape=[jax.ShapeDtypeStruct((N_NODES, D), jnp.float32),
                   jax.ShapeDtypeStruct((N_NODES, D), jnp.float32),
                   jax.ShapeDtypeStruct((N_NODES, D), jnp.float32),
                   jax.ShapeDtypeStruct((N_NODES, D), jnp.float32)],
    )(x, pW1, pb1, pW2, pb2, fW1, fb1, fW2, fb2, cW1ab, cb1ab)


def _combine_body(S_ref, C_ref, F_ref, aS_ref, aC_ref,
                  oW1_ref, ob1_ref, oW2_ref, ob2_ref, out_ref):
    agg = C_ref[...] * aS_ref[...] - S_ref[...] * aC_ref[...]
    pd = F_ref[...] + agg
    h = jnp.maximum(jnp.dot(pd, oW1_ref[...], precision=_HI,
                            preferred_element_type=jnp.float32) + ob1_ref[...], 0.0)
    out_ref[...] = jnp.dot(h, oW2_ref[...], precision=_HI,
                           preferred_element_type=jnp.float32) + ob2_ref[...]


def _combine(S, C, F, accS, accC, oW1, ob1, oW2, ob2):
    n_blocks = N_NODES // ROWS
    full = lambda s: pl.BlockSpec(s, lambda i: (0, 0))
    row = lambda w: pl.BlockSpec((ROWS, w), lambda i: (i, 0))
    return pl.pallas_call(
        _combine_body,
        grid=(n_blocks,),
        in_specs=[row(D), row(D), row(D), row(D), row(D),
                  full((D, H)), full((1, H)), full((H, D)), full((1, D))],
        out_specs=row(D),
        out_shape=jax.ShapeDtypeStruct((N_NODES, D), jnp.float32),
    )(S, C, F, accS, accC, oW1, ob1, oW2, ob2)


N_CHUNKS = EDGES_PER_SUB // CHUNK


def _sc_body(src_hbm, dst_hbm, T2_hbm, AB_hbm, w2_hbm, cb2_hbm,
             zero_hbm, outS_hbm, outC_hbm,
             isx, isd, trow, brow, arow, orow, w2v, cb2v, acc,
             isem, gsem, ssem):
    cid = lax.axis_index("c")
    sid = lax.axis_index("s")

    r0 = sid * ROWS_PER_SUB
    pltpu.sync_copy(zero_hbm.at[pl.ds(r0, ROWS_PER_SUB)],
                    acc.at[pl.ds(r0, ROWS_PER_SUB)])
    pltpu.sync_copy(w2_hbm, w2v)
    pltpu.sync_copy(cb2_hbm, cb2v)
    plsc.subcore_barrier()

    w2r = [w2v[pl.ds(16 * j, 16)] for j in range(H // 16)]
    cb2r = cb2v[...]

    def idx_list(j, r):
        return [(src_hbm.at[sid].at[j], isx.at[r]),
                (dst_hbm.at[sid].at[j], isd.at[r])]

    def issue_idx(j, r):
        for s, d in idx_list(j, r):
            pltpu.async_copy(s, d, isem.at[r])

    def wait_idx(j, r):
        for s, d in idx_list(j, r):
            pltpu.make_async_copy(s, d, isem.at[r]).wait()

    def gather_list(r, slot):
        si = isx.at[r]
        return [(T2_hbm.at[cid].at[si], trow.at[slot]),
                (AB_hbm.at[si], brow.at[slot]),
                (AB_hbm.at[isd.at[r]], arow.at[slot])]

    def issue_gathers(r, slot):
        for s, d in gather_list(r, slot):
            pltpu.async_copy(s, d, gsem.at[slot])

    def wait_gathers(r, slot):
        for s, d in gather_list(r, slot):
            pltpu.make_async_copy(s, d, gsem.at[slot]).wait()

    def compute_chunk(r, slot):
        @pl.loop(0, CHUNK, step=8)
        def _edge(e0):
            for u in range(8):
                e = e0 + u
                accv = cb2r
                for j in range(H // 16):
                    hj = jnp.maximum(arow[slot, e, pl.ds(16 * j, 16)]
                                     + brow[slot, e, pl.ds(H + 16 * j, 16)], 0.0)
                    accv = accv + hj * w2r[j]
                t = jnp.sum(accv)
                tv = jnp.full((16,), t, jnp.float32)
                cv = 1.0 / (1.0 + jnp.exp(-tv))
                for j in range(D // 16):
                    orow[slot, e, pl.ds(16 * j, 16)] = (
                        trow[slot, e, pl.ds(16 * j, 16)] * cv)

    def wait_scatter(r, slot):
        pltpu.make_async_copy(orow.at[slot], acc.at[isd.at[r]], ssem).wait()

    issue_idx(0, 0)
    issue_idx(1, 1)
    issue_idx(2, 2)
    wait_idx(0, 0)
    issue_gathers(0, 0)

    @pl.loop(0, N_CHUNKS, step=4)
    def _edge_chunk(j0):
        for u in range(4):
            j = j0 + u
            slot = u % 2

            @pl.when(j >= 1)
            def _():
                wait_scatter((u - 1) % 4, 1 - slot)

            @pl.when(j + 3 < N_CHUNKS)
            def _():
                issue_idx(j + 3, (u + 3) % 4)

            @pl.when(j + 1 < N_CHUNKS)
            def _():
                wait_idx(j + 1, (u + 1) % 4)
                issue_gathers((u + 1) % 4, 1 - slot)

            wait_gathers(u, slot)
            compute_chunk(u, slot)
            pltpu.async_copy(orow.at[slot], acc.at[isd.at[u]], ssem, add=True)

    wait_scatter((N_CHUNKS - 1) % 4, (N_CHUNKS - 1) % 2)
    plsc.subcore_barrier()

    @pl.when(cid == 0)
    def _():
        pltpu.sync_copy(acc.at[pl.ds(r0, ROWS_PER_SUB)],
                        outS_hbm.at[pl.ds(r0, ROWS_PER_SUB)])

    @pl.when(cid != 0)
    def _():
        pltpu.sync_copy(acc.at[pl.ds(r0, ROWS_PER_SUB)],
                        outC_hbm.at[pl.ds(r0, ROWS_PER_SUB)])


def _edge_scatter(src3, dst3, T2, AB, w2v, cb2v, zeros):
    mesh = plsc.VectorSubcoreMesh(core_axis_name="c", subcore_axis_name="s")
    f32 = jnp.float32
    kern = pl.kernel(
        _sc_body,
        out_type=(jax.ShapeDtypeStruct((N_PAD, D), f32),
                  jax.ShapeDtypeStruct((N_PAD, D), f32)),
        mesh=mesh,
        compiler_params=pltpu.CompilerParams(needs_layout_passes=False),
        scratch_types=[
            pltpu.VMEM((4, CHUNK), jnp.int32),
            pltpu.VMEM((4, CHUNK), jnp.int32),
            pltpu.VMEM((2, CHUNK, D), f32),
            pltpu.VMEM((2, CHUNK, D), f32),
            pltpu.VMEM((2, CHUNK, D), f32),
            pltpu.VMEM((2, CHUNK, D), f32),
            pltpu.VMEM((H,), f32),
            pltpu.VMEM((16,), f32),
            pltpu.VMEM_SHARED((N_PAD, D), f32),
            pltpu.SemaphoreType.DMA((4,)),
            pltpu.SemaphoreType.DMA((2,)),
            pltpu.SemaphoreType.DMA,
        ],
    )
    return kern(src3, dst3, T2, AB, w2v, cb2v, zeros)


def kernel(x, edge_index, pW1, pb1, pW2, pb2, fW1, fb1, fW2, fb2,
           cW1, cb1, cW2, cb2, oW1, ob1, oW2, ob2):
    src = edge_index[0].astype(jnp.int32)
    dst = edge_index[1].astype(jnp.int32)
    cW1ab = jnp.concatenate([cW1[:D], cW1[D:]], axis=1)
    cb1ab = jnp.concatenate([cb1, jnp.zeros((H,), jnp.float32)]).reshape(1, D)

    S, C, F, AB = _node_precompute(
        x, pW1, pb1.reshape(1, H), pW2, pb2.reshape(1, D),
        fW1, fb1.reshape(1, H), fW2, fb2.reshape(1, D),
        cW1ab, cb1ab)

    w2v = cW2[:, 0]
    cb2v = jnp.full((16,), 1.0, jnp.float32) * (cb2[0] / 16.0)
    zeros = jnp.zeros((N_PAD, D), jnp.float32)
    src3 = src.reshape(NS, N_CHUNKS, CHUNK)
    dst3 = dst.reshape(NS, N_CHUNKS, CHUNK)
    T2 = jnp.stack([S, C])
    accS, accC = _edge_scatter(src3, dst3, T2, AB, w2v, cb2v, zeros)

    return _combine(S, C, F, accS, accC,
                    oW1, ob1.reshape(1, H), oW2, ob2.reshape(1, D))


# speedup vs baseline: 1.1026x; 1.1026x over previous
import jax
import jax.numpy as jnp
from jax import lax
from jax.experimental import pallas as pl
from jax.experimental.pallas import tpu as pltpu
from jax.experimental.pallas import tpu_sc as plsc

N_NODES = 10000
N_PAD = 10240
N_EDGES = 320000
D = 128
H = 64

ROWS = 1000
NS = 16
CHUNK = 40
EDGES_PER_SUB = N_EDGES // NS
ROWS_PER_SUB = N_PAD // NS

_HI = jax.lax.Precision.HIGHEST


def _node_body(x_ref, pW1_ref, pb1_ref, pW2_ref, pb2_ref,
               fW1_ref, fb1_ref, fW2_ref, fb2_ref,
               cW1ab_ref, cb1ab_ref,
               S_ref, C_ref, F_ref, AB_ref):
    xb = x_ref[...]
    hp = jnp.tanh(jnp.dot(xb, pW1_ref[...], precision=_HI,
                          preferred_element_type=jnp.float32) + pb1_ref[...])
    phases = jnp.dot(hp, pW2_ref[...], precision=_HI,
                     preferred_element_type=jnp.float32) + pb2_ref[...]
    S_ref[...] = jnp.sin(phases)
    C_ref[...] = jnp.cos(phases)
    hf = jnp.maximum(jnp.dot(xb, fW1_ref[...], precision=_HI,
                             preferred_element_type=jnp.float32) + fb1_ref[...], 0.0)
    F_ref[...] = jnp.dot(hf, fW2_ref[...], precision=_HI,
                         preferred_element_type=jnp.float32) + fb2_ref[...]
    AB_ref[...] = jnp.dot(xb, cW1ab_ref[...], precision=_HI,
                          preferred_element_type=jnp.float32) + cb1ab_ref[...]


def _node_precompute(x, pW1, pb1, pW2, pb2, fW1, fb1, fW2, fb2, cW1ab, cb1ab):
    n_blocks = N_NODES // ROWS
    full = lambda s: pl.BlockSpec(s, lambda i: (0, 0))
    row = lambda w: pl.BlockSpec((ROWS, w), lambda i: (i, 0))
    return pl.pallas_call(
        _node_body,
        grid=(n_blocks,),
        in_specs=[row(D), full((D, H)), full((1, H)), full((H, D)), full((1, D)),
                  full((D, H)), full((1, H)), full((H, D)), full((1, D)),
                  full((D, D)), full((1, D))],
        out_specs=[row(D), row(D), row(D), row(D)],
        out_shape=[jax.ShapeDtypeStruct((N_NODES, D), jnp.float32),
                   jax.ShapeDtypeStruct((N_NODES, D), jnp.float32),
                   jax.ShapeDtypeStruct((N_NODES, D), jnp.float32),
                   jax.ShapeDtypeStruct((N_NODES, D), jnp.float32)],
    )(x, pW1, pb1, pW2, pb2, fW1, fb1, fW2, fb2, cW1ab, cb1ab)


def _combine_body(S_ref, C_ref, F_ref, aS_ref, aC_ref,
                  oW1_ref, ob1_ref, oW2_ref, ob2_ref, out_ref):
    agg = C_ref[...] * aS_ref[...] - S_ref[...] * aC_ref[...]
    pd = F_ref[...] + agg
    h = jnp.maximum(jnp.dot(pd, oW1_ref[...], precision=_HI,
                            preferred_element_type=jnp.float32) + ob1_ref[...], 0.0)
    out_ref[...] = jnp.dot(h, oW2_ref[...], precision=_HI,
                           preferred_element_type=jnp.float32) + ob2_ref[...]


def _combine(S, C, F, accS, accC, oW1, ob1, oW2, ob2):
    n_blocks = N_NODES // ROWS
    full = lambda s: pl.BlockSpec(s, lambda i: (0, 0))
    row = lambda w: pl.BlockSpec((ROWS, w), lambda i: (i, 0))
    return pl.pallas_call(
        _combine_body,
        grid=(n_blocks,),
        in_specs=[row(D), row(D), row(D), row(D), row(D),
                  full((D, H)), full((1, H)), full((H, D)), full((1, D))],
        out_specs=row(D),
        out_shape=jax.ShapeDtypeStruct((N_NODES, D), jnp.float32),
    )(S, C, F, accS, accC, oW1, ob1, oW2, ob2)


N_CHUNKS = EDGES_PER_SUB // CHUNK


def _sc_body(src_hbm, dst_hbm, T2_hbm, AB_hbm, w2_hbm, cb2_hbm,
             zero_hbm, outS_hbm, outC_hbm,
             isx, isd, trow, brow, arow, orow, w2v, cb2v, acc,
             isem, gsem, ssem):
    cid = lax.axis_index("c")
    sid = lax.axis_index("s")

    r0 = sid * ROWS_PER_SUB
    pltpu.sync_copy(zero_hbm.at[pl.ds(r0, ROWS_PER_SUB)],
                    acc.at[pl.ds(r0, ROWS_PER_SUB)])
    pltpu.sync_copy(w2_hbm, w2v)
    pltpu.sync_copy(cb2_hbm, cb2v)
    plsc.subcore_barrier()

    w2r = [w2v[pl.ds(16 * j, 16)] for j in range(H // 16)]
    cb2r = cb2v[...]

    def idx_list(j, r):
        return [(src_hbm.at[sid].at[j], isx.at[r]),
                (dst_hbm.at[sid].at[j], isd.at[r])]

    def issue_idx(j, r):
        for s, d in idx_list(j, r):
            pltpu.async_copy(s, d, isem.at[r])

    def wait_idx(j, r):
        for s, d in idx_list(j, r):
            pltpu.make_async_copy(s, d, isem.at[r]).wait()

    def gather_list(r, slot):
        si = isx.at[r]
        return [(T2_hbm.at[cid].at[si], trow.at[slot]),
                (AB_hbm.at[si], brow.at[slot]),
                (AB_hbm.at[isd.at[r]], arow.at[slot])]

    def issue_gathers(r, slot):
        for s, d in gather_list(r, slot):
            pltpu.async_copy(s, d, gsem.at[slot])

    def wait_gathers(r, slot):
        for s, d in gather_list(r, slot):
            pltpu.make_async_copy(s, d, gsem.at[slot]).wait()

    def compute_chunk(r, slot):
        @pl.loop(0, CHUNK, step=4)
        def _edge(e0):
            for u in range(4):
                e = e0 + u
                accv = cb2r
                for j in range(H // 16):
                    hj = jnp.maximum(arow[slot, e, pl.ds(16 * j, 16)]
                                     + brow[slot, e, pl.ds(H + 16 * j, 16)], 0.0)
                    accv = accv + hj * w2r[j]
                t = jnp.sum(accv)
                tv = jnp.full((16,), t, jnp.float32)
                cv = 1.0 / (1.0 + jnp.exp(-tv))
                for j in range(D // 16):
                    orow[slot, e, pl.ds(16 * j, 16)] = (
                        trow[slot, e, pl.ds(16 * j, 16)] * cv)

    def wait_scatter(r, slot):
        pltpu.make_async_copy(orow.at[slot], acc.at[isd.at[r]], ssem).wait()

    issue_idx(0, 0)
    issue_idx(1, 1)
    issue_idx(2, 2)
    wait_idx(0, 0)
    issue_gathers(0, 0)

    @pl.loop(0, N_CHUNKS, step=4)
    def _edge_chunk(j0):
        for u in range(4):
            j = j0 + u
            slot = u % 2

            @pl.when(j >= 1)
            def _():
                wait_scatter((u - 1) % 4, 1 - slot)

            @pl.when(j + 3 < N_CHUNKS)
            def _():
                issue_idx(j + 3, (u + 3) % 4)

            @pl.when(j + 1 < N_CHUNKS)
            def _():
                wait_idx(j + 1, (u + 1) % 4)
                issue_gathers((u + 1) % 4, 1 - slot)

            wait_gathers(u, slot)
            compute_chunk(u, slot)
            pltpu.async_copy(orow.at[slot], acc.at[isd.at[u]], ssem, add=True)

    wait_scatter((N_CHUNKS - 1) % 4, (N_CHUNKS - 1) % 2)
    plsc.subcore_barrier()

    @pl.when(cid == 0)
    def _():
        pltpu.sync_copy(acc.at[pl.ds(r0, ROWS_PER_SUB)],
                        outS_hbm.at[pl.ds(r0, ROWS_PER_SUB)])

    @pl.when(cid != 0)
    def _():
        pltpu.sync_copy(acc.at[pl.ds(r0, ROWS_PER_SUB)],
                        outC_hbm.at[pl.ds(r0, ROWS_PER_SUB)])


def _edge_scatter(src3, dst3, T2, AB, w2v, cb2v, zeros):
    mesh = plsc.VectorSubcoreMesh(core_axis_name="c", subcore_axis_name="s")
    f32 = jnp.float32
    kern = pl.kernel(
        _sc_body,
        out_type=(jax.ShapeDtypeStruct((N_PAD, D), f32),
                  jax.ShapeDtypeStruct((N_PAD, D), f32)),
        mesh=mesh,
        compiler_params=pltpu.CompilerParams(needs_layout_passes=False),
        scratch_types=[
            pltpu.VMEM((4, CHUNK), jnp.int32),
            pltpu.VMEM((4, CHUNK), jnp.int32),
            pltpu.VMEM((2, CHUNK, D), f32),
            pltpu.VMEM((2, CHUNK, D), f32),
            pltpu.VMEM((2, CHUNK, D), f32),
            pltpu.VMEM((2, CHUNK, D), f32),
            pltpu.VMEM((H,), f32),
            pltpu.VMEM((16,), f32),
            pltpu.VMEM_SHARED((N_PAD, D), f32),
            pltpu.SemaphoreType.DMA((4,)),
            pltpu.SemaphoreType.DMA((2,)),
            pltpu.SemaphoreType.DMA,
        ],
    )
    return kern(src3, dst3, T2, AB, w2v, cb2v, zeros)


def kernel(x, edge_index, pW1, pb1, pW2, pb2, fW1, fb1, fW2, fb2,
           cW1, cb1, cW2, cb2, oW1, ob1, oW2, ob2):
    src = edge_index[0].astype(jnp.int32)
    dst = edge_index[1].astype(jnp.int32)
    cW1ab = jnp.concatenate([cW1[:D], cW1[D:]], axis=1)
    cb1ab = jnp.concatenate([cb1, jnp.zeros((H,), jnp.float32)]).reshape(1, D)

    S, C, F, AB = _node_precompute(
        x, pW1, pb1.reshape(1, H), pW2, pb2.reshape(1, D),
        fW1, fb1.reshape(1, H), fW2, fb2.reshape(1, D),
        cW1ab, cb1ab)

    w2v = cW2[:, 0]
    cb2v = jnp.full((16,), 1.0, jnp.float32) * (cb2[0] / 16.0)
    zeros = jnp.zeros((N_PAD, D), jnp.float32)
    src3 = src.reshape(NS, N_CHUNKS, CHUNK)
    dst3 = dst.reshape(NS, N_CHUNKS, CHUNK)
    T2 = jnp.stack([S, C])
    accS, accC = _edge_scatter(src3, dst3, T2, AB, w2v, cb2v, zeros)

    return _combine(S, C, F, accS, accC,
                    oW1, ob1.reshape(1, H), oW2, ob2.reshape(1, D))
